# baseline (speedup 1.0000x reference)
import jax
import jax.numpy as jnp
from jax import lax
from jax.experimental import pallas as pl
from jax.experimental.pallas import tpu as pltpu
from jax.experimental.pallas import tpu_sc as plsc

N = 10000
D = 128
E = 320000
NC = 2
NS = 16
NW = NC * NS
EPW = E // NW
CH = 80
NCHUNK = EPW // CH
ACC_ROWS = 10240
RPT = ACC_ROWS // NS
MB = 1000


def _fuse_body(p_ref, w_ref, b_ref, o_ref):
    s = p_ref[0] + p_ref[1]
    o_ref[...] = jnp.maximum(
        jnp.dot(s, w_ref[...], preferred_element_type=jnp.float32)
        + b_ref[...], 0.0)


def _fuse(partials, W, b2):
    return pl.pallas_call(
        _fuse_body,
        grid=(N // MB,),
        in_specs=[pl.BlockSpec((NC, MB, D), lambda i: (0, i, 0)),
                  pl.BlockSpec((D, D), lambda i: (0, 0)),
                  pl.BlockSpec((1, D), lambda i: (0, 0))],
        out_specs=pl.BlockSpec((MB, D), lambda i: (i, 0)),
        out_shape=jax.ShapeDtypeStruct((N, D), jnp.float32),
    )(partials, W, b2)


def _scale(rows32_ref, rowsf_ref, val_ref):
    def group(g, c):
        base = g * 16
        v16 = val_ref[pl.ds(base, 16)]
        for l in range(16):
            vv = v16[l]
            e = base + l
            for k in range(D // 32):
                w = rows32_ref[e, pl.ds(k * 16, 16)]
                lo = lax.bitcast_convert_type(w << 16, jnp.float32)
                hi = lax.bitcast_convert_type(w & jnp.int32(-65536), jnp.float32)
                rowsf_ref[e, pl.ds(32 * k, 16)] = lo * vv
                rowsf_ref[e, pl.ds(32 * k + 16, 16)] = hi * vv
        return c
    lax.fori_loop(0, CH // 16, group, 0)


def _sc_body(x_hbm, src_hbm, dst_hbm, val_hbm, out_hbm,
             src_v, dst_a, dst_b, val_a, val_b, rows32_a, rows32_b,
             rowsf_a, rowsf_b, acc_sh,
             ga, gb, ha, hb, ka, kb, sa, sb):
    cid = lax.axis_index("c")
    sid = lax.axis_index("s")
    wid = sid * NC + cid
    ebase = wid * EPW

    scp = pltpu.async_copy(src_hbm.at[wid], src_v, ga)

    def zrow(i, c):
        for k in range(D // 16):
            rowsf_a[i, pl.ds(k * 16, 16)] = jnp.zeros((16,), jnp.float32)
        return c
    lax.fori_loop(0, CH, zrow, 0)
    for m in range(RPT // CH):
        pltpu.sync_copy(rowsf_a, acc_sh.at[pl.ds(sid * RPT + m * CH, CH)])
    scp.wait()
    plsc.subcore_barrier()

    bufs = [
        (rows32_a, rowsf_a, val_a, dst_a, ga, ha, ka, sa),
        (rows32_b, rowsf_b, val_b, dst_b, gb, hb, kb, sb),
    ]

    def issue3(j, p):
        r32, rf, v, d, g, h, k, ss = bufs[p]
        off = ebase + j * CH
        pltpu.async_copy(val_hbm.at[pl.ds(off, CH)], v, h)
        pltpu.async_copy(dst_hbm.at[pl.ds(off, CH)], d, k)
        pltpu.async_copy(x_hbm.at[src_v.at[j]], r32, g)

    def wait3(p):
        r32, rf, v, d, g, h, k, ss = bufs[p]
        pltpu.make_async_copy(val_hbm.at[pl.ds(ebase, CH)], v, h).wait()
        pltpu.make_async_copy(dst_hbm.at[pl.ds(ebase, CH)], d, k).wait()
        pltpu.make_async_copy(x_hbm.at[src_v.at[0]], r32, g).wait()

    def drain_scatter(p):
        r32, rf, v, d, g, h, k, ss = bufs[p]
        pltpu.make_async_copy(rf, acc_sh.at[d], ss).wait()

    def step(j, p, drain):
        r32, rf, v, d, g, h, k, ss = bufs[p]
        wait3(p)
        if drain:
            drain_scatter(p)
        _scale(r32, rf, v)
        pltpu.async_copy(rf, acc_sh.at[d], ss, add=True)

        def prefetch():
            issue3(j + 2, p)
        if isinstance(j, int):
            if j + 2 < NCHUNK:
                prefetch()
        else:
            pl.when(j + 2 < NCHUNK)(prefetch)

    issue3(0, 0)
    issue3(1, 1)
    step(0, 0, drain=False)
    step(1, 1, drain=False)

    def pair(jj, c):
        j0 = 2 * jj
        step(j0, 0, drain=True)
        step(j0 + 1, 1, drain=True)
        return c
    lax.fori_loop(1, NCHUNK // 2, pair, 0)
    step(NCHUNK - 1, 0, drain=True)
    drain_scatter(1)
    drain_scatter(0)

    plsc.subcore_barrier()
    for m in range(RPT // CH):
        r0 = sid * RPT + m * CH
        pltpu.sync_copy(acc_sh.at[pl.ds(r0, CH)],
                        out_hbm.at[cid, pl.ds(r0, CH)])


_sc_call = pl.kernel(
    _sc_body,
    out_type=jax.ShapeDtypeStruct((NC, ACC_ROWS, D), jnp.float32),
    mesh=plsc.VectorSubcoreMesh(core_axis_name="c", subcore_axis_name="s"),
    compiler_params=pltpu.CompilerParams(use_tc_tiling_on_sc=False),
    scratch_types=[
        pltpu.VMEM((NCHUNK, CH), jnp.int32),
        pltpu.VMEM((CH,), jnp.int32),
        pltpu.VMEM((CH,), jnp.int32),
        pltpu.VMEM((CH,), jnp.float32),
        pltpu.VMEM((CH,), jnp.float32),
        pltpu.VMEM((CH, D // 2), jnp.int32),
        pltpu.VMEM((CH, D // 2), jnp.int32),
        pltpu.VMEM((CH, D), jnp.float32),
        pltpu.VMEM((CH, D), jnp.float32),
        pltpu.VMEM_SHARED((ACC_ROWS, D), jnp.float32),
    ] + [pltpu.SemaphoreType.DMA] * 8,
)


def kernel(x, lap_indices, lap_values, W, b):
    x16 = x.astype(jnp.bfloat16).reshape(N, D // 32, 2, 16)
    x32 = jax.lax.bitcast_convert_type(
        x16.transpose(0, 1, 3, 2), jnp.int32).reshape(N, D // 2)
    dst = lap_indices[0]
    src = lap_indices[1].reshape(NW, NCHUNK, CH)
    partials = _sc_call(x32, src, dst, lap_values)
    return _fuse(partials, W, b.reshape(1, D))


# speedup vs baseline: 1.8469x; 1.8469x over previous
import jax
import jax.numpy as jnp
from jax import lax
from jax.experimental import pallas as pl
from jax.experimental.pallas import tpu as pltpu
from jax.experimental.pallas import tpu_sc as plsc

N = 10000
D = 128
E = 320000
NC = 2
NS = 16
NW = NC * NS
EPW = E // NW
CH = 80
NCHUNK = EPW // CH
ACC_ROWS = 10240
RPT = ACC_ROWS // NS
MB = 1000


def _fuse_body(p_ref, w_ref, b_ref, o_ref):
    s = p_ref[0] + p_ref[1]
    o_ref[...] = jnp.maximum(
        jnp.dot(s, w_ref[...], preferred_element_type=jnp.float32)
        + b_ref[...], 0.0)


def _fuse(partials, W, b2):
    return pl.pallas_call(
        _fuse_body,
        grid=(N // MB,),
        in_specs=[pl.BlockSpec((NC, MB, D), lambda i: (0, i, 0)),
                  pl.BlockSpec((D, D), lambda i: (0, 0)),
                  pl.BlockSpec((1, D), lambda i: (0, 0))],
        out_specs=pl.BlockSpec((MB, D), lambda i: (i, 0)),
        out_shape=jax.ShapeDtypeStruct((N, D), jnp.float32),
    )(partials, W, b2)


def _scale(rows_ref, val_ref):
    def group(g, c):
        base = g * 16
        v16 = val_ref[pl.ds(base, 16)]
        for l in range(16):
            vv = v16[l]
            for k in range(D // 16):
                sl = pl.ds(k * 16, 16)
                rows_ref[base + l, sl] = rows_ref[base + l, sl] * vv
        return c
    lax.fori_loop(0, CH // 16, group, 0)


def _sc_body(x_hbm, src_hbm, dst_hbm, val_hbm, out_hbm,
             src_v, dst_a, dst_b, dst_c, val_a, val_b, val_c,
             rows_a, rows_b, rows_c, acc_sh,
             ga, gb, gc, ha, hb, hc, ka, kb, kc, sa, sb, sc):
    cid = lax.axis_index("c")
    sid = lax.axis_index("s")
    wid = sid * NC + cid
    ebase = wid * EPW

    scp = pltpu.async_copy(src_hbm.at[wid], src_v, ga)

    def zrow(i, c):
        for k in range(D // 16):
            rows_a[i, pl.ds(k * 16, 16)] = jnp.zeros((16,), jnp.float32)
        return c
    lax.fori_loop(0, CH, zrow, 0)
    for m in range(RPT // CH):
        pltpu.sync_copy(rows_a, acc_sh.at[pl.ds(sid * RPT + m * CH, CH)])
    scp.wait()
    plsc.subcore_barrier()

    def issue3(j, rows_ref, val_ref, dst_ref, gsem, hsem, ksem):
        off = ebase + j * CH
        pltpu.async_copy(val_hbm.at[pl.ds(off, CH)], val_ref, hsem)
        pltpu.async_copy(dst_hbm.at[pl.ds(off, CH)], dst_ref, ksem)
        pltpu.async_copy(x_hbm.at[src_v.at[j]], rows_ref, gsem)

    def wait3(rows_ref, val_ref, dst_ref, gsem, hsem, ksem):
        pltpu.make_async_copy(val_hbm.at[pl.ds(ebase, CH)], val_ref, hsem).wait()
        pltpu.make_async_copy(dst_hbm.at[pl.ds(ebase, CH)], dst_ref, ksem).wait()
        pltpu.make_async_copy(x_hbm.at[src_v.at[0]], rows_ref, gsem).wait()

    def drain_scatter(rows_ref, dst_ref, ssem):
        pltpu.make_async_copy(rows_ref, acc_sh.at[dst_ref], ssem).wait()

    bufs = [
        (rows_a, val_a, dst_a, ga, ha, ka, sa),
        (rows_b, val_b, dst_b, gb, hb, kb, sb),
        (rows_c, val_c, dst_c, gc, hc, kc, sc),
    ]

    def half_async(j, cur, nxt, first_prefetch=False):
        r, v, d, g, h, k, ss = bufs[cur]
        rn, vn, dn, gn, hn, kn, ssn = bufs[nxt]
        wait3(r, v, d, g, h, k)

        def prefetch():
            if not first_prefetch:
                drain_scatter(rn, dn, ssn)
            issue3(j + 2, rn, vn, dn, gn, hn, kn)
        if isinstance(j, int) and j + 2 < NCHUNK:
            prefetch()
        elif not isinstance(j, int):
            pl.when(j + 2 < NCHUNK)(prefetch)
        _scale(r, v)
        pltpu.async_copy(r, acc_sh.at[d], ss, add=True)

    issue3(0, rows_a, val_a, dst_a, ga, ha, ka)
    issue3(1, rows_b, val_b, dst_b, gb, hb, kb)
    half_async(0, 0, 2, first_prefetch=True)

    def group3(g, c):
        base = 3 * g
        half_async(base + 1, 1, 0)
        half_async(base + 2, 2, 1)
        half_async(base + 3, 0, 2)
        return c
    lax.fori_loop(0, (NCHUNK - 2) // 3, group3, 0)

    r, v, d, g, h, k, ss = bufs[1]
    wait3(r, v, d, g, h, k)
    _scale(r, v)
    pltpu.sync_copy(r, acc_sh.at[d], add=True)
    drain_scatter(rows_a, dst_a, sa)
    drain_scatter(rows_c, dst_c, sc)

    plsc.subcore_barrier()
    for m in range(RPT // CH):
        r0 = sid * RPT + m * CH
        pltpu.sync_copy(acc_sh.at[pl.ds(r0, CH)],
                        out_hbm.at[cid, pl.ds(r0, CH)])


_sc_call = pl.kernel(
    _sc_body,
    out_type=jax.ShapeDtypeStruct((NC, ACC_ROWS, D), jnp.float32),
    mesh=plsc.VectorSubcoreMesh(core_axis_name="c", subcore_axis_name="s"),
    scratch_types=[
        pltpu.VMEM((NCHUNK, CH), jnp.int32),
        pltpu.VMEM((CH,), jnp.int32),
        pltpu.VMEM((CH,), jnp.int32),
        pltpu.VMEM((CH,), jnp.int32),
        pltpu.VMEM((CH,), jnp.float32),
        pltpu.VMEM((CH,), jnp.float32),
        pltpu.VMEM((CH,), jnp.float32),
        pltpu.VMEM((CH, D), jnp.float32),
        pltpu.VMEM((CH, D), jnp.float32),
        pltpu.VMEM((CH, D), jnp.float32),
        pltpu.VMEM_SHARED((ACC_ROWS, D), jnp.float32),
    ] + [pltpu.SemaphoreType.DMA] * 12,
)


def kernel(x, lap_indices, lap_values, W, b):
    dst = lap_indices[0]
    src = lap_indices[1].reshape(NW, NCHUNK, CH)
    partials = _sc_call(x, src, dst, lap_values)
    return _fuse(partials, W, b.reshape(1, D))


# speedup vs baseline: 1.8527x; 1.0031x over previous
import jax
import jax.numpy as jnp
from jax import lax
from jax.experimental import pallas as pl
from jax.experimental.pallas import tpu as pltpu
from jax.experimental.pallas import tpu_sc as plsc

N = 10000
D = 128
E = 320000
NC = 2
NS = 16
NW = NC * NS
EPW = E // NW
CH = 80
NCHUNK = EPW // CH
ACC_ROWS = 10240
RPT = ACC_ROWS // NS
MB = 1000


def _fuse_body(p_ref, w_ref, b_ref, o_ref):
    s = p_ref[0] + p_ref[1]
    o_ref[...] = jnp.maximum(
        jnp.dot(s, w_ref[...], preferred_element_type=jnp.float32)
        + b_ref[...], 0.0)


def _fuse(partials, W, b2):
    return pl.pallas_call(
        _fuse_body,
        grid=(N // MB,),
        in_specs=[pl.BlockSpec((NC, MB, D), lambda i: (0, i, 0)),
                  pl.BlockSpec((D, D), lambda i: (0, 0)),
                  pl.BlockSpec((1, D), lambda i: (0, 0))],
        out_specs=pl.BlockSpec((MB, D), lambda i: (i, 0)),
        out_shape=jax.ShapeDtypeStruct((N, D), jnp.float32),
    )(partials, W, b2)


def _scale(rows_ref, val_ref):
    def group(g, c):
        base = g * 16
        v16 = val_ref[pl.ds(base, 16)]
        for l in range(16):
            vv = v16[l]
            for k in range(D // 16):
                sl = pl.ds(k * 16, 16)
                rows_ref[base + l, sl] = rows_ref[base + l, sl] * vv
        return c
    lax.fori_loop(0, CH // 16, group, 0)


def _sc_body(x_hbm, src_hbm, dst_hbm, val_hbm, out_hbm,
             src_v, dst_a, dst_b, dst_c, val_a, val_b, val_c,
             rows_a, rows_b, rows_c, acc_sh,
             ga, gb, gc, ha, hb, hc, ka, kb, kc, sa, sb, sc):
    cid = lax.axis_index("c")
    sid = lax.axis_index("s")
    wid = sid * NC + cid
    ebase = wid * EPW

    scp = pltpu.async_copy(src_hbm.at[wid], src_v, ga)

    def zrow(i, c):
        for k in range(D // 16):
            rows_a[i, pl.ds(k * 16, 16)] = jnp.zeros((16,), jnp.float32)
        return c
    lax.fori_loop(0, CH, zrow, 0)
    zsems = [gb, gc, ha, hb, hc, ka, kb, kc]
    for m in range(RPT // CH):
        pltpu.async_copy(rows_a, acc_sh.at[pl.ds(sid * RPT + m * CH, CH)],
                         zsems[m])
    for m in range(RPT // CH):
        pltpu.make_async_copy(rows_a, acc_sh.at[pl.ds(sid * RPT + m * CH, CH)],
                              zsems[m]).wait()
    scp.wait()
    plsc.subcore_barrier()

    def issue3(j, rows_ref, val_ref, dst_ref, gsem, hsem, ksem):
        off = ebase + j * CH
        pltpu.async_copy(val_hbm.at[pl.ds(off, CH)], val_ref, hsem)
        pltpu.async_copy(dst_hbm.at[pl.ds(off, CH)], dst_ref, ksem)
        pltpu.async_copy(x_hbm.at[src_v.at[j]], rows_ref, gsem)

    def wait3(rows_ref, val_ref, dst_ref, gsem, hsem, ksem):
        pltpu.make_async_copy(val_hbm.at[pl.ds(ebase, CH)], val_ref, hsem).wait()
        pltpu.make_async_copy(dst_hbm.at[pl.ds(ebase, CH)], dst_ref, ksem).wait()
        pltpu.make_async_copy(x_hbm.at[src_v.at[0]], rows_ref, gsem).wait()

    def drain_scatter(rows_ref, dst_ref, ssem):
        pltpu.make_async_copy(rows_ref, acc_sh.at[dst_ref], ssem).wait()

    bufs = [
        (rows_a, val_a, dst_a, ga, ha, ka, sa),
        (rows_b, val_b, dst_b, gb, hb, kb, sb),
        (rows_c, val_c, dst_c, gc, hc, kc, sc),
    ]

    def half_async(j, cur, nxt, first_prefetch=False):
        r, v, d, g, h, k, ss = bufs[cur]
        rn, vn, dn, gn, hn, kn, ssn = bufs[nxt]
        wait3(r, v, d, g, h, k)

        def prefetch():
            if not first_prefetch:
                drain_scatter(rn, dn, ssn)
            issue3(j + 2, rn, vn, dn, gn, hn, kn)
        if isinstance(j, int) and j + 2 < NCHUNK:
            prefetch()
        elif not isinstance(j, int):
            pl.when(j + 2 < NCHUNK)(prefetch)
        _scale(r, v)
        pltpu.async_copy(r, acc_sh.at[d], ss, add=True)

    issue3(0, rows_a, val_a, dst_a, ga, ha, ka)
    issue3(1, rows_b, val_b, dst_b, gb, hb, kb)
    half_async(0, 0, 2, first_prefetch=True)

    def group3(g, c):
        base = 3 * g
        half_async(base + 1, 1, 0)
        half_async(base + 2, 2, 1)
        half_async(base + 3, 0, 2)
        return c
    lax.fori_loop(0, (NCHUNK - 2) // 3, group3, 0)

    r, v, d, g, h, k, ss = bufs[1]
    wait3(r, v, d, g, h, k)
    _scale(r, v)
    pltpu.sync_copy(r, acc_sh.at[d], add=True)
    drain_scatter(rows_a, dst_a, sa)
    drain_scatter(rows_c, dst_c, sc)

    plsc.subcore_barrier()
    wsems = [ga, gb, gc, ha, hb, hc, ka, kb]
    for m in range(RPT // CH):
        r0 = sid * RPT + m * CH
        pltpu.async_copy(acc_sh.at[pl.ds(r0, CH)],
                         out_hbm.at[cid, pl.ds(r0, CH)], wsems[m])
    for m in range(RPT // CH):
        r0 = sid * RPT + m * CH
        pltpu.make_async_copy(acc_sh.at[pl.ds(r0, CH)],
                              out_hbm.at[cid, pl.ds(r0, CH)], wsems[m]).wait()


_sc_call = pl.kernel(
    _sc_body,
    out_type=jax.ShapeDtypeStruct((NC, ACC_ROWS, D), jnp.float32),
    mesh=plsc.VectorSubcoreMesh(core_axis_name="c", subcore_axis_name="s"),
    scratch_types=[
        pltpu.VMEM((NCHUNK, CH), jnp.int32),
        pltpu.VMEM((CH,), jnp.int32),
        pltpu.VMEM((CH,), jnp.int32),
        pltpu.VMEM((CH,), jnp.int32),
        pltpu.VMEM((CH,), jnp.float32),
        pltpu.VMEM((CH,), jnp.float32),
        pltpu.VMEM((CH,), jnp.float32),
        pltpu.VMEM((CH, D), jnp.float32),
        pltpu.VMEM((CH, D), jnp.float32),
        pltpu.VMEM((CH, D), jnp.float32),
        pltpu.VMEM_SHARED((ACC_ROWS, D), jnp.float32),
    ] + [pltpu.SemaphoreType.DMA] * 12,
)


def kernel(x, lap_indices, lap_values, W, b):
    dst = lap_indices[0]
    src = lap_indices[1].reshape(NW, NCHUNK, CH)
    partials = _sc_call(x, src, dst, lap_values)
    return _fuse(partials, W, b.reshape(1, D))
